# baseline (speedup 1.0000x reference)
import functools

import jax
import jax.numpy as jnp
from jax import lax
from jax.experimental import pallas as pl
from jax.experimental.pallas import tpu as pltpu
from jax.experimental.pallas import tpu_sc as plsc

N = 320000
D = 128
G = 1024
HID = 128

NC = 2
NS = 16
RCHUNKS = 16
HALVES = 2
DH = D // HALVES
RPW = N // RCHUNKS
C = 160
NCH = RPW // C
GPC = C // 16
CL = 16


def _seg_pool_sc(x, ids):
    mesh = plsc.VectorSubcoreMesh(core_axis_name="c", subcore_axis_name="s")

    @functools.partial(
        pl.kernel,
        out_type=(
            jax.ShapeDtypeStruct((RCHUNKS, G, D), jnp.float32),
            jax.ShapeDtypeStruct((RCHUNKS, G, CL), jnp.float32),
        ),
        mesh=mesh,
        compiler_params=pltpu.CompilerParams(use_tc_tiling_on_sc=False, needs_layout_passes=False),
        scratch_types=(
            pltpu.VMEM((G, DH), jnp.float32),
            pltpu.VMEM((G, CL), jnp.float32),
            pltpu.VMEM((2, C, DH), jnp.float32),
            pltpu.VMEM((RPW,), jnp.int32),
            pltpu.SemaphoreType.DMA,
            pltpu.SemaphoreType.DMA,
            pltpu.SemaphoreType.DMA,
        ),
    )
    def seg_kernel(x_hbm, ids_hbm, sums_hbm, cnts_hbm,
                   acc, cacc, xbuf, idbuf, xs0, xs1, isem):
        cid = lax.axis_index("c")
        sid = lax.axis_index("s")
        iota16 = lax.iota(jnp.int32, 16)
        cols = [iota16 + 16 * j for j in range(DH // 16)]
        ones16 = jnp.full((16,), 1.0, jnp.float32)
        zeros16 = jnp.full((16,), 0.0, jnp.float32)
        row0 = sid * RPW
        col0 = cid * DH
        xsem = (xs0, xs1)

        def dma_x(chunk, b):
            base = row0 + chunk * C
            return pltpu.make_async_copy(
                x_hbm.at[pl.ds(base, C), pl.ds(col0, DH)], xbuf.at[b], xsem[b])

        def dma_ids():
            return pltpu.make_async_copy(
                ids_hbm.at[pl.ds(row0, RPW)], idbuf, isem)

        @pl.loop(0, G)
        def _zero(i):
            risp = jnp.full((16,), i, jnp.int32)
            for j in range(DH // 16):
                plsc.store_scatter(acc, [risp, cols[j]], zeros16)
            plsc.store_scatter(cacc, [risp, cols[0]], zeros16)

        def bcast(vec16, lane):
            dn = lax.GatherDimensionNumbers(
                offset_dims=(), collapsed_slice_dims=(0,), start_index_map=(0,))
            return lax.gather(
                vec16, jnp.full((16, 1), lane, jnp.int32), dn,
                slice_sizes=(1,),
                mode=lax.GatherScatterMode.PROMISE_IN_BOUNDS)

        def tree_sum(vals):
            while len(vals) > 1:
                nxt = [vals[k] + vals[k + 1] for k in range(0, len(vals) - 1, 2)]
                if len(vals) % 2:
                    nxt.append(vals[-1])
                vals = nxt
            return vals[0]

        def process(chunk, b):
            @pl.loop(0, GPC)
            def _group(g):
                base = g * 16
                ids16 = jnp.minimum(
                    idbuf[pl.ds(chunk * C + base, 16)], G - 1)
                plsc.addupdate_scatter(cacc, [ids16, iota16], ones16)
                idsp0 = bcast(ids16, 0)
                uniform = ids16[0] == ids16[15]

                @pl.when(uniform)
                def _fast():
                    for j in range(DH // 16):
                        sj = tree_sum(
                            [xbuf[b, base + i, pl.ds(16 * j, 16)]
                             for i in range(16)])
                        plsc.addupdate_scatter(acc, [idsp0, cols[j]], sj)

                @pl.when(jnp.logical_not(uniform))
                def _slow():
                    for i in range(16):
                        idsp = bcast(ids16, i)
                        for j in range(DH // 16):
                            v = xbuf[b, base + i, pl.ds(16 * j, 16)]
                            plsc.addupdate_scatter(acc, [idsp, cols[j]], v)

        dma_ids().start()
        dma_x(0, 0).start()
        dma_x(1, 1).start()
        dma_ids().wait()

        @pl.loop(0, NCH // 2)
        def _chunk_pair(i2):
            for b in range(2):
                chunk = i2 * 2 + b
                dma_x(chunk, b).wait()
                process(chunk, b)
                nxt = chunk + 2

                @pl.when(nxt < NCH)
                def _start_next():
                    dma_x(nxt, b).start()

        dma_x(NCH - 1, 0).wait()
        process(NCH - 1, 0)

        pltpu.sync_copy(acc, sums_hbm.at[sid, :, pl.ds(col0, DH)])

        @pl.when(cid == 0)
        def _write_counts():
            pltpu.sync_copy(cacc, cnts_hbm.at[sid])

    return seg_kernel(x, ids)


def _head_tc(sums_part, cnt_part, W1, b1, W2, b2):
    def body(sp_ref, cp_ref, w1_ref, b1_ref, w2_ref, b2_ref, out_ref):
        sum_pool = jnp.sum(sp_ref[...], axis=0)
        counts = jnp.sum(cp_ref[...], axis=(0, 2))
        counts = jnp.maximum(counts, 1.0)
        mean_pool = sum_pool / counts[:, None]
        w1a = w1_ref[pl.ds(0, D), :]
        w1b = w1_ref[pl.ds(D, D), :]
        h1 = (jnp.dot(sum_pool, w1a, preferred_element_type=jnp.float32)
              + jnp.dot(mean_pool, w1b, preferred_element_type=jnp.float32)
              + b1_ref[...])
        h1 = jnp.maximum(h1, 0.0)
        out_ref[...] = (jnp.dot(h1, w2_ref[...],
                                preferred_element_type=jnp.float32)
                        + b2_ref[...])

    return pl.pallas_call(
        body,
        out_shape=jax.ShapeDtypeStruct((G, 1), jnp.float32),
    )(sums_part, cnt_part, W1, b1, W2, b2)


def kernel(x, segment_ids, num_graphs, W1, b1, W2, b2):
    ids = segment_ids.astype(jnp.int32)
    sums_part, cnt_part = _seg_pool_sc(x, ids)
    return _head_tc(sums_part, cnt_part, W1, b1, W2, b2)


# speedup vs baseline: 1.0140x; 1.0140x over previous
import functools

import jax
import jax.numpy as jnp
from jax import lax
from jax.experimental import pallas as pl
from jax.experimental.pallas import tpu as pltpu
from jax.experimental.pallas import tpu_sc as plsc

N = 320000
D = 128
G = 1024
HID = 128

NC = 2
NS = 16
RCHUNKS = 16
HALVES = 2
DH = D // HALVES
RPW = N // RCHUNKS
C = 160
NCH = RPW // C
GPC = C // 16
CL = 16


def _seg_pool_sc(x, ids):
    mesh = plsc.VectorSubcoreMesh(core_axis_name="c", subcore_axis_name="s")

    @functools.partial(
        pl.kernel,
        out_type=(
            jax.ShapeDtypeStruct((RCHUNKS, G, D), jnp.float32),
            jax.ShapeDtypeStruct((RCHUNKS, G, CL), jnp.float32),
        ),
        mesh=mesh,
        compiler_params=pltpu.CompilerParams(use_tc_tiling_on_sc=False, needs_layout_passes=False),
        scratch_types=(
            pltpu.VMEM((G, DH), jnp.float32),
            pltpu.VMEM((G, CL), jnp.float32),
            pltpu.VMEM((2, C, DH), jnp.float32),
            pltpu.VMEM((RPW,), jnp.int32),
            pltpu.SemaphoreType.DMA,
            pltpu.SemaphoreType.DMA,
            pltpu.SemaphoreType.DMA,
        ),
    )
    def seg_kernel(x_hbm, ids_hbm, sums_hbm, cnts_hbm,
                   acc, cacc, xbuf, idbuf, xs0, xs1, isem):
        cid = lax.axis_index("c")
        sid = lax.axis_index("s")
        iota16 = lax.iota(jnp.int32, 16)
        cols = [iota16 + 16 * j for j in range(DH // 16)]
        ones16 = jnp.full((16,), 1.0, jnp.float32)
        zeros16 = jnp.full((16,), 0.0, jnp.float32)
        row0 = sid * RPW
        col0 = cid * DH
        xsem = (xs0, xs1)

        def dma_x(chunk, b):
            base = row0 + chunk * C
            return pltpu.make_async_copy(
                x_hbm.at[pl.ds(base, C), pl.ds(col0, DH)], xbuf.at[b], xsem[b])

        def dma_ids():
            return pltpu.make_async_copy(
                ids_hbm.at[pl.ds(row0, RPW)], idbuf, isem)

        def bcast(vec16, lane):
            dn = lax.GatherDimensionNumbers(
                offset_dims=(), collapsed_slice_dims=(0,), start_index_map=(0,))
            return lax.gather(
                vec16, jnp.full((16, 1), lane, jnp.int32), dn,
                slice_sizes=(1,),
                mode=lax.GatherScatterMode.PROMISE_IN_BOUNDS)

        def tree_sum(vals):
            while len(vals) > 1:
                nxt = [vals[k] + vals[k + 1] for k in range(0, len(vals) - 1, 2)]
                if len(vals) % 2:
                    nxt.append(vals[-1])
                vals = nxt
            return vals[0]

        def process(chunk, b):
            @pl.loop(0, GPC)
            def _group(g):
                base = g * 16
                ids16 = jnp.minimum(
                    idbuf[pl.ds(chunk * C + base, 16)], G - 1)
                plsc.addupdate_scatter(cacc, [ids16, iota16], ones16)
                idsp0 = bcast(ids16, 0)
                uniform = ids16[0] == ids16[15]

                @pl.when(uniform)
                def _fast():
                    for j in range(DH // 16):
                        sj = tree_sum(
                            [xbuf[b, base + i, pl.ds(16 * j, 16)]
                             for i in range(16)])
                        plsc.addupdate_scatter(acc, [idsp0, cols[j]], sj)

                @pl.when(jnp.logical_not(uniform))
                def _slow():
                    for i in range(16):
                        idsp = bcast(ids16, i)
                        for j in range(DH // 16):
                            v = xbuf[b, base + i, pl.ds(16 * j, 16)]
                            plsc.addupdate_scatter(acc, [idsp, cols[j]], v)

        dma_ids().start()
        dma_x(0, 0).start()
        dma_x(1, 1).start()

        @pl.loop(0, G)
        def _zero(i):
            risp = jnp.full((16,), i, jnp.int32)
            for j in range(DH // 16):
                plsc.store_scatter(acc, [risp, cols[j]], zeros16)
            plsc.store_scatter(cacc, [risp, cols[0]], zeros16)

        dma_ids().wait()

        @pl.loop(0, NCH // 2)
        def _chunk_pair(i2):
            for b in range(2):
                chunk = i2 * 2 + b
                dma_x(chunk, b).wait()
                process(chunk, b)
                nxt = chunk + 2

                @pl.when(nxt < NCH)
                def _start_next():
                    dma_x(nxt, b).start()

        dma_x(NCH - 1, 0).wait()
        process(NCH - 1, 0)

        pltpu.sync_copy(acc, sums_hbm.at[sid, :, pl.ds(col0, DH)])

        @pl.when(cid == 0)
        def _write_counts():
            pltpu.sync_copy(cacc, cnts_hbm.at[sid])

    return seg_kernel(x, ids)


def _head_tc(sums_part, cnt_part, W1, b1, W2, b2):
    def body(sp_ref, cp_ref, w1_ref, b1_ref, w2_ref, b2_ref, out_ref):
        sum_pool = jnp.sum(sp_ref[...], axis=0)
        counts = jnp.sum(cp_ref[...], axis=(0, 2))
        counts = jnp.maximum(counts, 1.0)
        mean_pool = sum_pool / counts[:, None]
        w1a = w1_ref[pl.ds(0, D), :]
        w1b = w1_ref[pl.ds(D, D), :]
        h1 = (jnp.dot(sum_pool, w1a, preferred_element_type=jnp.float32)
              + jnp.dot(mean_pool, w1b, preferred_element_type=jnp.float32)
              + b1_ref[...])
        h1 = jnp.maximum(h1, 0.0)
        out_ref[...] = (jnp.dot(h1, w2_ref[...],
                                preferred_element_type=jnp.float32)
                        + b2_ref[...])

    return pl.pallas_call(
        body,
        out_shape=jax.ShapeDtypeStruct((G, 1), jnp.float32),
    )(sums_part, cnt_part, W1, b1, W2, b2)


def kernel(x, segment_ids, num_graphs, W1, b1, W2, b2):
    ids = segment_ids.astype(jnp.int32)
    sums_part, cnt_part = _seg_pool_sc(x, ids)
    return _head_tc(sums_part, cnt_part, W1, b1, W2, b2)
